# Initial kernel scaffold; baseline (speedup 1.0000x reference)
import jax
import jax.numpy as jnp
from jax.experimental import pallas as pl


def kernel(entity_emb, relation_emb, weight, pos_triplets, neg_triplets, edge_index, edge_type):
    raise NotImplementedError("write your pallas kernel here")


# speedup vs baseline: 4.9243x; 4.9243x over previous
import functools

import jax
import jax.numpy as jnp
from jax import lax
from jax.experimental import pallas as pl
from jax.experimental.pallas import tpu as pltpu
from jax.experimental.pallas import tpu_sc as plsc

NC = 2
NS = 16
LANES = 16
NW = NC * NS


def _transform_body(e_ref, w_ref, tr_ref):
    tr_ref[0] = jnp.dot(e_ref[...], w_ref[0], preferred_element_type=jnp.float32)


def _transform(entity_emb, weight):
    n, d = entity_emb.shape
    r = weight.shape[0]
    nblk = 5
    bn = n // nblk
    tr = pl.pallas_call(
        _transform_body,
        grid=(nblk, r),
        in_specs=[
            pl.BlockSpec((bn, d), lambda i, j: (i, 0)),
            pl.BlockSpec((1, d, d), lambda i, j: (j, 0, 0)),
        ],
        out_specs=pl.BlockSpec((1, bn, d), lambda i, j: (j, i, 0)),
        out_shape=jax.ShapeDtypeStruct((r, n, d), jnp.float32),
    )(entity_emb, weight)
    return tr.reshape(r * n, d)


def _make_scatter(n, d, e, chunk):
    mesh = plsc.VectorSubcoreMesh(core_axis_name="c", subcore_axis_name="s", num_cores=NC, num_subcores=NS)
    echunks = e // chunk
    rbase = (n // NS) & ~7
    zstep = 16

    @functools.partial(
        pl.kernel,
        out_type=jax.ShapeDtypeStruct((NC, n, d), jnp.float32),
        mesh=mesh,
        scratch_types=[
            pltpu.VMEM_SHARED((n, d), jnp.float32),
            pltpu.VMEM((chunk,), jnp.int32),
            pltpu.VMEM((chunk,), jnp.int32),
            pltpu.VMEM((chunk,), jnp.int32),
            pltpu.VMEM((chunk,), jnp.int32),
            pltpu.VMEM((chunk, d), jnp.float32),
            pltpu.VMEM((zstep, d), jnp.float32),
            pltpu.SemaphoreType.DMA,
        ],
    )
    def scatter_k(tr_hbm, src_hbm, tgt_hbm, et_hbm, out_hbm,
                  acc, srcb, etb, idxb, tgtb, rowsb, zbuf, sem):
        cid = lax.axis_index("c")
        sid = lax.axis_index("s")
        wid = sid * NC + cid

        base = sid * rbase
        myrows = jnp.where(sid == NS - 1, n - (NS - 1) * rbase, rbase)
        nz = myrows // zstep

        for i in range(zstep):
            for s in range(d // LANES):
                zbuf[i, pl.ds(s * LANES, LANES)] = jnp.zeros((LANES,), jnp.float32)

        def _zcopy(z, carry):
            pltpu.sync_copy(zbuf, acc.at[pl.ds(base + z * zstep, zstep)])
            return carry
        lax.fori_loop(0, nz, _zcopy, 0)
        plsc.subcore_barrier()

        nfull = echunks // NW
        nchunks = jnp.where(wid < echunks % NW, nfull + 1, nfull)

        def _edge_chunk(j, carry):
            off = (wid + j * NW) * chunk
            pltpu.sync_copy(src_hbm.at[pl.ds(off, chunk)], srcb)
            pltpu.sync_copy(et_hbm.at[pl.ds(off, chunk)], etb)
            pltpu.sync_copy(tgt_hbm.at[pl.ds(off, chunk)], tgtb)
            for s in range(chunk // LANES):
                sl = pl.ds(s * LANES, LANES)
                idxb[sl] = etb[sl] * n + srcb[sl]
            pltpu.async_copy(tr_hbm.at[idxb], rowsb, sem).wait()
            pltpu.sync_copy(rowsb, acc.at[tgtb], add=True)
            return carry
        lax.fori_loop(0, nchunks, _edge_chunk, 0)
        plsc.subcore_barrier()

        def _wcopy(z, carry):
            pltpu.sync_copy(acc.at[pl.ds(base + z * zstep, zstep)],
                            out_hbm.at[cid, pl.ds(base + z * zstep, zstep)])
            return carry
        lax.fori_loop(0, nz, _wcopy, 0)

    return scatter_k


def _combine_body(p_ref, u_ref):
    u_ref[...] = jnp.maximum(p_ref[0] + p_ref[1], 0.0)


def _combine(partial):
    _, n, d = partial.shape
    nblk = 5
    bn = n // nblk
    return pl.pallas_call(
        _combine_body,
        grid=(nblk,),
        in_specs=[pl.BlockSpec((2, bn, d), lambda i: (0, i, 0))],
        out_specs=pl.BlockSpec((bn, d), lambda i: (i, 0)),
        out_shape=jax.ShapeDtypeStruct((n, d), jnp.float32),
    )(partial)


def _make_score(n, d, r, t, chunk):
    mesh = plsc.VectorSubcoreMesh(core_axis_name="c", subcore_axis_name="s", num_cores=NC, num_subcores=NS)
    tchunks = t // chunk
    rows_per_w = tchunks // NW
    pb = chunk * LANES

    @functools.partial(
        pl.kernel,
        out_type=(jax.ShapeDtypeStruct((t * LANES,), jnp.float32),
                  jax.ShapeDtypeStruct((t * LANES,), jnp.float32)),
        mesh=mesh,
        scratch_types=[
            pltpu.VMEM((chunk,), jnp.int32),
            pltpu.VMEM((chunk,), jnp.int32),
            pltpu.VMEM((chunk,), jnp.int32),
            pltpu.VMEM((chunk, d), jnp.float32),
            pltpu.VMEM((chunk, d), jnp.float32),
            pltpu.VMEM((chunk, d), jnp.float32),
            pltpu.VMEM((pb,), jnp.float32),
            pltpu.SemaphoreType.DMA,
        ],
    )
    def score_k(upd_hbm, rel_hbm, ph, pr, pt, nh, nr, nt, pos_out, neg_out,
                hidx, ridx, tidx, hrows, rrows, trows_b, partb, sem):
        cid = lax.axis_index("c")
        sid = lax.axis_index("s")
        wid = sid * NC + cid

        for (hsrc, rsrc, tsrc, out_hbm) in ((ph, pr, pt, pos_out),
                                            (nh, nr, nt, neg_out)):
            def _row(q, carry, hsrc=hsrc, rsrc=rsrc, tsrc=tsrc, out_hbm=out_hbm):
                rowid = wid * rows_per_w + q
                toff = rowid * chunk
                pltpu.sync_copy(hsrc.at[pl.ds(toff, chunk)], hidx)
                pltpu.sync_copy(rsrc.at[pl.ds(toff, chunk)], ridx)
                pltpu.sync_copy(tsrc.at[pl.ds(toff, chunk)], tidx)
                pltpu.async_copy(upd_hbm.at[hidx], hrows, sem).wait()
                pltpu.async_copy(rel_hbm.at[ridx], rrows, sem).wait()
                pltpu.async_copy(upd_hbm.at[tidx], trows_b, sem).wait()

                def _trip(i, c2):
                    acc = jnp.zeros((LANES,), jnp.float32)
                    for s in range(d // LANES):
                        sl = pl.ds(s * LANES, LANES)
                        dd = hrows[i, sl] + rrows[i, sl] - trows_b[i, sl]
                        acc = acc + dd * dd
                    partb[pl.ds(i * LANES, LANES)] = acc
                    return c2
                lax.fori_loop(0, chunk, _trip, 0)
                pltpu.sync_copy(partb, out_hbm.at[pl.ds(rowid * pb, pb)])
                return carry
            lax.fori_loop(0, rows_per_w, _row, 0)

    return score_k


def _loss_body(pp_ref, out_ref):
    p = pp_ref[0]
    q = pp_ref[1]
    ps = jnp.sqrt(jnp.sum(p, axis=1, keepdims=True))
    ns = jnp.sqrt(jnp.sum(q, axis=1, keepdims=True))
    hinge = jnp.maximum(ps - ns + 1.0, 0.0)
    out_ref[...] = jnp.sum(hinge, axis=(0, 1), keepdims=True) / p.shape[0]


def _loss(parts):
    return pl.pallas_call(
        _loss_body,
        in_specs=[pl.BlockSpec(parts.shape, lambda: (0, 0, 0))],
        out_specs=pl.BlockSpec((1, 1), lambda: (0, 0)),
        out_shape=jax.ShapeDtypeStruct((1, 1), jnp.float32),
    )(parts)


def kernel(entity_emb, relation_emb, weight, pos_triplets, neg_triplets,
           edge_index, edge_type):
    n, d = entity_emb.shape
    r = weight.shape[0]
    e = edge_type.shape[0]
    t = pos_triplets.shape[0]
    chunk = 128

    tr_flat = _transform(entity_emb, weight)

    partial = _make_scatter(n, d, e, chunk)(
        tr_flat, edge_index[0], edge_index[1], edge_type)

    updated = _combine(partial)

    pos_out, neg_out = _make_score(n, d, r, t, chunk)(
        updated, relation_emb,
        pos_triplets[:, 0], pos_triplets[:, 1], pos_triplets[:, 2],
        neg_triplets[:, 0], neg_triplets[:, 1], neg_triplets[:, 2])
    parts = jnp.stack([pos_out.reshape(t, LANES), neg_out.reshape(t, LANES)])

    loss = _loss(parts)
    return loss[0, 0]
